# baseline (device time: 43345 ns/iter reference)
import jax
import jax.numpy as jnp
from jax import lax
from jax.experimental import pallas as pl
from jax.experimental.pallas import tpu as pltpu

B, S, D = 2, 256, 1024
DC_SH = 64
H, DH, DR = 16, 64, 32
SCALE = (DH + DR) ** -0.5


def kernel(x, Wdkv, Wuk, Wuv, Wq, Wqr, Wkr, Wo):
    def body(x_ref, wdkv_ref, wuk_ref, wuv_ref, wq_ref, wqr_ref, wkr_ref,
             wo_ref, out_ref,
             cm, cp, wukp, wuvp, q, qr, kr, k, v, o,
             send_sems, recv_sems):
        my_x = lax.axis_index("x")
        my_y = lax.axis_index("y")
        peer = (my_x, 1 - my_y)

        barrier = pltpu.get_barrier_semaphore()
        pl.semaphore_signal(barrier, inc=1, device_id=peer,
                            device_id_type=pl.DeviceIdType.MESH)
        pl.semaphore_wait(barrier, 1)

        wuk_rdma = pltpu.make_async_remote_copy(
            src_ref=wuk_ref, dst_ref=wukp,
            send_sem=send_sems.at[0], recv_sem=recv_sems.at[0],
            device_id=peer, device_id_type=pl.DeviceIdType.MESH)
        wuk_rdma.start()
        wuv_rdma = pltpu.make_async_remote_copy(
            src_ref=wuv_ref, dst_ref=wuvp,
            send_sem=send_sems.at[1], recv_sem=recv_sems.at[1],
            device_id=peer, device_id_type=pl.DeviceIdType.MESH)
        wuv_rdma.start()

        for b in range(B):
            cm[b, :, :] = jnp.dot(x_ref[b, :, :], wdkv_ref[...],
                                  preferred_element_type=jnp.float32)
        c_rdma = pltpu.make_async_remote_copy(
            src_ref=cm, dst_ref=cp,
            send_sem=send_sems.at[2], recv_sem=recv_sems.at[2],
            device_id=peer, device_id_type=pl.DeviceIdType.MESH)
        c_rdma.start()

        for b in range(B):
            xb = x_ref[b, :, :]
            q[b, :, :] = jnp.dot(xb, wq_ref[...],
                                 preferred_element_type=jnp.float32)
            qr[b, :, :] = jnp.dot(xb, wqr_ref[...],
                                  preferred_element_type=jnp.float32)
            kr[b, :, :] = jnp.dot(xb, wkr_ref[...],
                                  preferred_element_type=jnp.float32)

        wuk_rdma.wait()
        wuv_rdma.wait()
        c_rdma.wait()

        for b in range(B):
            k[b, :, :] = (jnp.dot(cm[b, :, :], wuk_ref[...],
                                  preferred_element_type=jnp.float32)
                          + jnp.dot(cp[b, :, :], wukp[...],
                                    preferred_element_type=jnp.float32))
            v[b, :, :] = (jnp.dot(cm[b, :, :], wuv_ref[...],
                                  preferred_element_type=jnp.float32)
                          + jnp.dot(cp[b, :, :], wuvp[...],
                                    preferred_element_type=jnp.float32))

        for b in range(B):
            krb = kr[b, :, :]
            for h in range(H):
                qh = q[b, :, h * DH:(h + 1) * DH]
                kh = k[b, :, h * DH:(h + 1) * DH]
                vh = v[b, :, h * DH:(h + 1) * DH]
                qrh = qr[b, :, h * DR:(h + 1) * DR]
                sc = (lax.dot_general(qh, kh, (((1,), (1,)), ((), ())),
                                      preferred_element_type=jnp.float32)
                      + lax.dot_general(qrh, krb, (((1,), (1,)), ((), ())),
                                        preferred_element_type=jnp.float32)
                      ) * SCALE
                m = jnp.max(sc, axis=1, keepdims=True)
                p = jnp.exp(sc - m)
                p = p / jnp.sum(p, axis=1, keepdims=True)
                o[b, :, h * DH:(h + 1) * DH] = jnp.dot(
                    p, vh, preferred_element_type=jnp.float32)

        for b in range(B):
            out_ref[b, :, :] = jnp.dot(o[b, :, :], wo_ref[...],
                                       preferred_element_type=jnp.float32)

    return pl.pallas_call(
        body,
        out_shape=jax.ShapeDtypeStruct((B, S, D), jnp.float32),
        in_specs=[pl.BlockSpec(memory_space=pltpu.VMEM)] * 8,
        out_specs=pl.BlockSpec(memory_space=pltpu.VMEM),
        scratch_shapes=[
            pltpu.VMEM((B, S, DC_SH), jnp.float32),
            pltpu.VMEM((B, S, DC_SH), jnp.float32),
            pltpu.VMEM((DC_SH, D), jnp.float32),
            pltpu.VMEM((DC_SH, D), jnp.float32),
            pltpu.VMEM((B, S, D), jnp.float32),
            pltpu.VMEM((B, S, H * DR), jnp.float32),
            pltpu.VMEM((B, S, DR), jnp.float32),
            pltpu.VMEM((B, S, D), jnp.float32),
            pltpu.VMEM((B, S, D), jnp.float32),
            pltpu.VMEM((B, S, D), jnp.float32),
            pltpu.SemaphoreType.DMA((3,)),
            pltpu.SemaphoreType.DMA((3,)),
        ],
        compiler_params=pltpu.CompilerParams(collective_id=0),
    )(x, Wdkv, Wuk, Wuv, Wq, Wqr, Wkr, Wo)


# device time: 43225 ns/iter; 1.0028x vs baseline; 1.0028x over previous
import jax
import jax.numpy as jnp
from jax import lax
from jax.experimental import pallas as pl
from jax.experimental.pallas import tpu as pltpu

B, S, D = 2, 256, 1024
M = B * S
DC_SH = 64
DC = 2 * DC_SH
H, DH, DR = 16, 64, 32
SCALE = (DH + DR) ** -0.5


def kernel(x, Wdkv, Wuk, Wuv, Wq, Wqr, Wkr, Wo):
    def body(x_ref, wdkv_ref, wuk_ref, wuv_ref, wq_ref, wqr_ref, wkr_ref,
             wo_ref, out_ref,
             cm, cp, ccat, wukcat, wuvcat, q, qr, kr, k, v, o,
             send_sems, recv_sems):
        my_x = lax.axis_index("x")
        my_y = lax.axis_index("y")
        peer = (my_x, 1 - my_y)

        barrier = pltpu.get_barrier_semaphore()
        pl.semaphore_signal(barrier, inc=1, device_id=peer,
                            device_id_type=pl.DeviceIdType.MESH)
        pl.semaphore_wait(barrier, 1)

        wuk_rdma = pltpu.make_async_remote_copy(
            src_ref=wuk_ref, dst_ref=wukcat.at[pl.ds(DC_SH, DC_SH), :],
            send_sem=send_sems.at[0], recv_sem=recv_sems.at[0],
            device_id=peer, device_id_type=pl.DeviceIdType.MESH)
        wuk_rdma.start()
        wuv_rdma = pltpu.make_async_remote_copy(
            src_ref=wuv_ref, dst_ref=wuvcat.at[pl.ds(DC_SH, DC_SH), :],
            send_sem=send_sems.at[1], recv_sem=recv_sems.at[1],
            device_id=peer, device_id_type=pl.DeviceIdType.MESH)
        wuv_rdma.start()

        x2 = jnp.reshape(x_ref[...], (M, D))

        c_mine = jnp.dot(x2, wdkv_ref[...], preferred_element_type=jnp.float32)
        cm[...] = c_mine
        ccat[:, 0:DC_SH] = c_mine
        c_rdma = pltpu.make_async_remote_copy(
            src_ref=cm, dst_ref=cp,
            send_sem=send_sems.at[2], recv_sem=recv_sems.at[2],
            device_id=peer, device_id_type=pl.DeviceIdType.MESH)
        c_rdma.start()

        q[...] = jnp.dot(x2, wq_ref[...], preferred_element_type=jnp.float32)
        qr[...] = jnp.dot(x2, wqr_ref[...], preferred_element_type=jnp.float32)
        kr[...] = jnp.dot(x2, wkr_ref[...], preferred_element_type=jnp.float32)
        wukcat[0:DC_SH, :] = wuk_ref[...]
        wuvcat[0:DC_SH, :] = wuv_ref[...]

        wuk_rdma.wait()
        wuv_rdma.wait()
        c_rdma.wait()

        ccat[:, DC_SH:DC] = cp[...]

        k[...] = jnp.dot(ccat[...], wukcat[...],
                         preferred_element_type=jnp.float32)
        v[...] = jnp.dot(ccat[...], wuvcat[...],
                         preferred_element_type=jnp.float32)

        for b in range(B):
            r = slice(b * S, (b + 1) * S)
            krb = kr[r, :]
            for h in range(H):
                qh = q[r, h * DH:(h + 1) * DH]
                kh = k[r, h * DH:(h + 1) * DH]
                vh = v[r, h * DH:(h + 1) * DH]
                qrh = qr[r, h * DR:(h + 1) * DR]
                sc = (lax.dot_general(qh, kh, (((1,), (1,)), ((), ())),
                                      preferred_element_type=jnp.float32)
                      + lax.dot_general(qrh, krb, (((1,), (1,)), ((), ())),
                                        preferred_element_type=jnp.float32)
                      ) * SCALE
                m = jnp.max(sc, axis=1, keepdims=True)
                p = jnp.exp(sc - m)
                p = p / jnp.sum(p, axis=1, keepdims=True)
                o[r, h * DH:(h + 1) * DH] = jnp.dot(
                    p, vh, preferred_element_type=jnp.float32)

        out2 = jnp.dot(o[...], wo_ref[...], preferred_element_type=jnp.float32)
        out_ref[...] = jnp.reshape(out2, (B, S, D))

    return pl.pallas_call(
        body,
        out_shape=jax.ShapeDtypeStruct((B, S, D), jnp.float32),
        in_specs=[pl.BlockSpec(memory_space=pltpu.VMEM)] * 8,
        out_specs=pl.BlockSpec(memory_space=pltpu.VMEM),
        scratch_shapes=[
            pltpu.VMEM((M, DC_SH), jnp.float32),
            pltpu.VMEM((M, DC_SH), jnp.float32),
            pltpu.VMEM((M, DC), jnp.float32),
            pltpu.VMEM((DC, D), jnp.float32),
            pltpu.VMEM((DC, D), jnp.float32),
            pltpu.VMEM((M, D), jnp.float32),
            pltpu.VMEM((M, H * DR), jnp.float32),
            pltpu.VMEM((M, DR), jnp.float32),
            pltpu.VMEM((M, D), jnp.float32),
            pltpu.VMEM((M, D), jnp.float32),
            pltpu.VMEM((M, D), jnp.float32),
            pltpu.SemaphoreType.DMA((3,)),
            pltpu.SemaphoreType.DMA((3,)),
        ],
        compiler_params=pltpu.CompilerParams(collective_id=0),
    )(x, Wdkv, Wuk, Wuv, Wq, Wqr, Wkr, Wo)
